# baseline (speedup 1.0000x reference)
import jax
import jax.numpy as jnp
from jax import lax
from jax.experimental import pallas as pl
from jax.experimental.pallas import tpu as pltpu
from jax.experimental.pallas import tpu_sc as plsc

_N = 10000
_E = 320000
_FIN = 128
_HID = 64
_NCLS = 40

_NC = 2
_NS = 16
_NW = _NC * _NS
_NPAD = 10240
_SL = _NPAD // _NS
_K = 128
_NCHUNK = _E // _K
_FULL = _NCHUNK // _NW
_REM = _NCHUNK - _FULL * _NW


def _mesh():
  return plsc.VectorSubcoreMesh(
      core_axis_name="c", subcore_axis_name="s", num_cores=_NC,
      num_subcores=_NS)


_SC_PARAMS = pltpu.CompilerParams(use_tc_tiling_on_sc=False)


_DK = 26


def _deg_body(ei_hbm, degp_hbm, dbuf, onesv, zbuf, acc, sem):
  cid = lax.axis_index("c")
  sid = lax.axis_index("s")
  wid = cid * _NS + sid

  pltpu.sync_copy(ei_hbm.at[1, pl.ds(wid * _FULL, _FULL)],
                  dbuf.at[pl.ds(0, _FULL)])

  @pl.when(wid < _REM)
  def _():
    pltpu.sync_copy(ei_hbm.at[1, pl.ds(_NW * _FULL + wid, 1)],
                    dbuf.at[pl.ds(_FULL, 1)])

  def _fill_z(i, c):
    zbuf[pl.ds(i * 16, 16)] = jnp.zeros((16,), jnp.float32)
    return c
  lax.fori_loop(0, _SL // 16, _fill_z, 0)

  def _fill_o(i, c):
    onesv[pl.ds(i * 16, 16)] = jnp.ones((16,), jnp.float32)
    return c
  lax.fori_loop(0, _K // 16, _fill_o, 0)

  pltpu.sync_copy(zbuf, acc.at[pl.ds(sid * _SL, _SL)])
  plsc.subcore_barrier()

  def _round(r, c):
    def _fire(j, c2):
      pltpu.async_copy(onesv, acc.at[dbuf.at[r * _DK + j]], sem, add=True)
      return c2
    lax.fori_loop(0, _DK, _fire, 0)

    def _drain(j, c2):
      pltpu.make_async_copy(onesv, acc.at[dbuf.at[r * _DK + j]], sem).wait()
      return c2
    lax.fori_loop(0, _DK, _drain, 0)
    return c
  lax.fori_loop(0, _FULL // _DK, _round, 0)

  @pl.when(wid < _REM)
  def _():
    pltpu.sync_copy(onesv, acc.at[dbuf.at[_FULL]], add=True)

  plsc.subcore_barrier()
  pltpu.sync_copy(acc.at[pl.ds(sid * _SL, _SL)],
                  degp_hbm.at[cid, pl.ds(sid * _SL, _SL)])


_deg_call = pl.kernel(
    _deg_body,
    out_type=jax.ShapeDtypeStruct((_NC, _NPAD), jnp.float32),
    mesh=_mesh(),
    compiler_params=_SC_PARAMS,
    scratch_types=[
        pltpu.VMEM((_FULL + 1, _K), jnp.int32),
        pltpu.VMEM((_K,), jnp.float32),
        pltpu.VMEM((_SL,), jnp.float32),
        pltpu.VMEM_SHARED((_NPAD,), jnp.float32),
        pltpu.SemaphoreType.DMA,
    ],
)


def _make_agg(d, _NBUF, _LOOK):
  _SKIP = _NBUF - _LOOK
  _TRIPS = _FULL // _NBUF

  def _agg_body(h_hbm, ei_hbm, outp, sbuf, dbuf, rows, zrows, acc, *sems):
    cid = lax.axis_index("c")
    sid = lax.axis_index("s")
    wid = cid * _NS + sid
    gsem = list(sems[:_NBUF])
    ssem = list(sems[_NBUF:])
    base = wid * _FULL

    ld_s = pltpu.async_copy(ei_hbm.at[0, pl.ds(base, _FULL)],
                            sbuf.at[pl.ds(0, _FULL)], gsem[0])
    ld_d = pltpu.async_copy(ei_hbm.at[1, pl.ds(base, _FULL)],
                            dbuf.at[pl.ds(0, _FULL)], gsem[1])

    @pl.when(wid < _REM)
    def _():
      pltpu.sync_copy(ei_hbm.at[0, pl.ds(_NW * _FULL + wid, 1)],
                      sbuf.at[pl.ds(_FULL, 1)])
      pltpu.sync_copy(ei_hbm.at[1, pl.ds(_NW * _FULL + wid, 1)],
                      dbuf.at[pl.ds(_FULL, 1)])

    offs = list(range(0, d - 15, 16))
    if d % 16 != 0:
      offs.append(d - 16)
    for i in range(32):
      for j in offs:
        zrows[i, pl.ds(j, 16)] = jnp.zeros((16,), jnp.float32)

    def _zero(t, c):
      pltpu.sync_copy(zrows, acc.at[pl.ds(sid * _SL + t * 32, 32)])
      return c
    lax.fori_loop(0, _SL // 32, _zero, 0)
    ld_s.wait()
    ld_d.wait()
    plsc.subcore_barrier()

    for b in range(_LOOK):
      pltpu.async_copy(h_hbm.at[sbuf.at[b]], rows.at[b], gsem[b])

    def _trip(t, carry):
      for b in range(_NBUF):
        c = t * _NBUF + b
        bg = (b + _LOOK) % _NBUF
        pltpu.make_async_copy(h_hbm.at[sbuf.at[c]], rows.at[b],
                              gsem[b]).wait()
        pltpu.async_copy(rows.at[b], acc.at[dbuf.at[c]], ssem[b], add=True)

        def _wait_old():
          pltpu.make_async_copy(rows.at[bg], acc.at[dbuf.at[c]],
                                ssem[bg]).wait()

        def _issue_gather():
          pltpu.async_copy(h_hbm.at[sbuf.at[c + _LOOK]], rows.at[bg],
                           gsem[bg])

        if b < _SKIP:
          @pl.when(t > 0)
          def _w():
            _wait_old()
          _issue_gather()
        else:
          _wait_old()

          @pl.when(t < _TRIPS - 1)
          def _g():
            _issue_gather()
      return carry
    lax.fori_loop(0, _TRIPS, _trip, 0)

    for b in range(_LOOK, _NBUF):
      pltpu.make_async_copy(rows.at[b], acc.at[dbuf.at[0]], ssem[b]).wait()

    @pl.when(wid < _REM)
    def _():
      pltpu.async_copy(h_hbm.at[sbuf.at[_FULL]], rows.at[0], gsem[0]).wait()
      pltpu.sync_copy(rows.at[0], acc.at[dbuf.at[_FULL]], add=True)

    plsc.subcore_barrier()
    pltpu.sync_copy(acc.at[pl.ds(sid * _SL, _SL)],
                    outp.at[cid, pl.ds(sid * _SL, _SL)])

  return pl.kernel(
      _agg_body,
      out_type=jax.ShapeDtypeStruct((_NC, _NPAD, d), jnp.float32),
      mesh=_mesh(),
      compiler_params=_SC_PARAMS,
      scratch_types=[
          pltpu.VMEM((_FULL + 1, _K), jnp.int32),
          pltpu.VMEM((_FULL + 1, _K), jnp.int32),
          pltpu.VMEM((_NBUF, _K, d), jnp.float32),
          pltpu.VMEM((32, d), jnp.float32),
          pltpu.VMEM_SHARED((_NPAD, d), jnp.float32),
      ] + [pltpu.SemaphoreType.DMA] * (2 * _NBUF),
  )


_agg64 = _make_agg(_HID, 6, 5)
_agg40 = _make_agg(_NCLS, 13, 11)


_BR = 1024
_GRID = (_N + _BR - 1) // _BR


def _mm1_body(x_ref, w_ref, degp_ref, o_ref):
  s = lax.rsqrt(degp_ref[0, :] + degp_ref[1, :] + 1.0)
  h = jnp.dot(x_ref[...], w_ref[...], preferred_element_type=jnp.float32)
  o_ref[...] = h * s[:, None]


def _mm1(x, w1, degp):
  return pl.pallas_call(
      _mm1_body,
      grid=(_GRID,),
      in_specs=[
          pl.BlockSpec((_BR, _FIN), lambda i: (i, 0)),
          pl.BlockSpec((_FIN, _HID), lambda i: (0, 0)),
          pl.BlockSpec((_NC, _BR), lambda i: (0, i)),
      ],
      out_specs=pl.BlockSpec((_BR, _HID), lambda i: (i, 0)),
      out_shape=jax.ShapeDtypeStruct((_NPAD, _HID), jnp.float32),
  )(x, w1, degp)


def _mm2_body(degp_ref, p_ref, h_ref, b_ref, w_ref, o_ref):
  s = lax.rsqrt(degp_ref[0, :] + degp_ref[1, :] + 1.0)
  t = (p_ref[0] + p_ref[1] + h_ref[...]) * s[:, None] + b_ref[...]
  z = jnp.maximum(t, 0.0)
  o_ref[...] = jnp.dot(
      z, w_ref[...], preferred_element_type=jnp.float32) * s[:, None]


def _mm2(degp, p, h1, b1r, w2):
  return pl.pallas_call(
      _mm2_body,
      grid=(_GRID,),
      in_specs=[
          pl.BlockSpec((_NC, _BR), lambda i: (0, i)),
          pl.BlockSpec((_NC, _BR, _HID), lambda i: (0, i, 0)),
          pl.BlockSpec((_BR, _HID), lambda i: (i, 0)),
          pl.BlockSpec((1, _HID), lambda i: (0, 0)),
          pl.BlockSpec((_HID, _NCLS), lambda i: (0, 0)),
      ],
      out_specs=pl.BlockSpec((_BR, _NCLS), lambda i: (i, 0)),
      out_shape=jax.ShapeDtypeStruct((_NPAD, _NCLS), jnp.float32),
  )(degp, p, h1, b1r, w2)


def _fin_body(degp_ref, p_ref, h_ref, b_ref, o_ref):
  s = lax.rsqrt(degp_ref[0, :] + degp_ref[1, :] + 1.0)
  u = (p_ref[0] + p_ref[1] + h_ref[...]) * s[:, None] + b_ref[...]
  m = jnp.max(u, axis=1, keepdims=True)
  lse = jnp.log(jnp.sum(jnp.exp(u - m), axis=1, keepdims=True)) + m
  o_ref[...] = u - lse


def _fin(degp, p, h2, b2r):
  return pl.pallas_call(
      _fin_body,
      grid=(_GRID,),
      in_specs=[
          pl.BlockSpec((_NC, _BR), lambda i: (0, i)),
          pl.BlockSpec((_NC, _BR, _NCLS), lambda i: (0, i, 0)),
          pl.BlockSpec((_BR, _NCLS), lambda i: (i, 0)),
          pl.BlockSpec((1, _NCLS), lambda i: (0, 0)),
      ],
      out_specs=pl.BlockSpec((_BR, _NCLS), lambda i: (i, 0)),
      out_shape=jax.ShapeDtypeStruct((_N, _NCLS), jnp.float32),
  )(degp, p, h2, b2r)


@jax.jit
def kernel(x, edge_index, W1, b1, W2, b2):
  ei3 = edge_index.astype(jnp.int32).reshape(2, _NCHUNK, _K)
  b1r = b1.reshape(1, _HID)
  b2r = b2.reshape(1, _NCLS)

  degp = _deg_call(ei3)
  h1 = _mm1(x, W1, degp)
  p1 = _agg64(h1, ei3)
  h2 = _mm2(degp, p1, h1, b1r, W2)
  p2 = _agg40(h2, ei3)
  return _fin(degp, p2, h2, b2r)


# speedup vs baseline: 1.0027x; 1.0027x over previous
import jax
import jax.numpy as jnp
from jax import lax
from jax.experimental import pallas as pl
from jax.experimental.pallas import tpu as pltpu
from jax.experimental.pallas import tpu_sc as plsc

_N = 10000
_E = 320000
_FIN = 128
_HID = 64
_NCLS = 40

_NC = 2
_NS = 16
_NW = _NC * _NS
_NPAD = 10240
_SL = _NPAD // _NS
_K = 128
_NCHUNK = _E // _K
_FULL = _NCHUNK // _NW
_REM = _NCHUNK - _FULL * _NW


def _mesh():
  return plsc.VectorSubcoreMesh(
      core_axis_name="c", subcore_axis_name="s", num_cores=_NC,
      num_subcores=_NS)


_SC_PARAMS = pltpu.CompilerParams(use_tc_tiling_on_sc=False)


_DK = 26


def _deg_body(ei_hbm, degp_hbm, dbuf, onesv, zbuf, acc, sem):
  cid = lax.axis_index("c")
  sid = lax.axis_index("s")
  wid = cid * _NS + sid

  pltpu.sync_copy(ei_hbm.at[1, pl.ds(wid * _FULL, _FULL)],
                  dbuf.at[pl.ds(0, _FULL)])

  @pl.when(wid < _REM)
  def _():
    pltpu.sync_copy(ei_hbm.at[1, pl.ds(_NW * _FULL + wid, 1)],
                    dbuf.at[pl.ds(_FULL, 1)])

  def _fill_z(i, c):
    zbuf[pl.ds(i * 16, 16)] = jnp.zeros((16,), jnp.float32)
    return c
  lax.fori_loop(0, _SL // 16, _fill_z, 0)

  def _fill_o(i, c):
    onesv[pl.ds(i * 16, 16)] = jnp.ones((16,), jnp.float32)
    return c
  lax.fori_loop(0, _K // 16, _fill_o, 0)

  pltpu.sync_copy(zbuf, acc.at[pl.ds(sid * _SL, _SL)])
  plsc.subcore_barrier()

  def _round(r, c):
    def _fire(j, c2):
      pltpu.async_copy(onesv, acc.at[dbuf.at[r * _DK + j]], sem, add=True)
      return c2
    lax.fori_loop(0, _DK, _fire, 0)

    def _drain(j, c2):
      pltpu.make_async_copy(onesv, acc.at[dbuf.at[r * _DK + j]], sem).wait()
      return c2
    lax.fori_loop(0, _DK, _drain, 0)
    return c
  lax.fori_loop(0, _FULL // _DK, _round, 0)

  @pl.when(wid < _REM)
  def _():
    pltpu.sync_copy(onesv, acc.at[dbuf.at[_FULL]], add=True)

  plsc.subcore_barrier()
  pltpu.sync_copy(acc.at[pl.ds(sid * _SL, _SL)],
                  degp_hbm.at[cid, pl.ds(sid * _SL, _SL)])


_deg_call = pl.kernel(
    _deg_body,
    out_type=jax.ShapeDtypeStruct((_NC, _NPAD), jnp.float32),
    mesh=_mesh(),
    compiler_params=_SC_PARAMS,
    scratch_types=[
        pltpu.VMEM((_FULL + 1, _K), jnp.int32),
        pltpu.VMEM((_K,), jnp.float32),
        pltpu.VMEM((_SL,), jnp.float32),
        pltpu.VMEM_SHARED((_NPAD,), jnp.float32),
        pltpu.SemaphoreType.DMA,
    ],
)


def _make_agg(d, _NBUF, _LOOK):
  _SKIP = _NBUF - _LOOK
  _TRIPS = _FULL // _NBUF

  def _agg_body(h_hbm, ei_hbm, outp, sbuf, dbuf, rows, zrows, acc, *sems):
    cid = lax.axis_index("c")
    sid = lax.axis_index("s")
    wid = cid * _NS + sid
    gsem = list(sems[:_NBUF])
    ssem = list(sems[_NBUF:])
    base = wid * _FULL

    ld_s = pltpu.async_copy(ei_hbm.at[0, pl.ds(base, _FULL)],
                            sbuf.at[pl.ds(0, _FULL)], gsem[0])
    ld_d = pltpu.async_copy(ei_hbm.at[1, pl.ds(base, _FULL)],
                            dbuf.at[pl.ds(0, _FULL)], gsem[1])

    @pl.when(wid < _REM)
    def _():
      pltpu.sync_copy(ei_hbm.at[0, pl.ds(_NW * _FULL + wid, 1)],
                      sbuf.at[pl.ds(_FULL, 1)])
      pltpu.sync_copy(ei_hbm.at[1, pl.ds(_NW * _FULL + wid, 1)],
                      dbuf.at[pl.ds(_FULL, 1)])

    offs = list(range(0, d - 15, 16))
    if d % 16 != 0:
      offs.append(d - 16)
    for i in range(32):
      for j in offs:
        zrows[i, pl.ds(j, 16)] = jnp.zeros((16,), jnp.float32)

    def _zero(t, c):
      pltpu.sync_copy(zrows, acc.at[pl.ds(sid * _SL + t * 32, 32)])
      return c
    lax.fori_loop(0, _SL // 32, _zero, 0)
    ld_s.wait()
    ld_d.wait()
    plsc.subcore_barrier()

    for b in range(_LOOK):
      pltpu.async_copy(h_hbm.at[sbuf.at[b]], rows.at[b], gsem[b])

    def _trip(t, carry):
      for b in range(_NBUF):
        c = t * _NBUF + b
        bg = (b + _LOOK) % _NBUF
        pltpu.make_async_copy(h_hbm.at[sbuf.at[c]], rows.at[b],
                              gsem[b]).wait()
        pltpu.async_copy(rows.at[b], acc.at[dbuf.at[c]], ssem[b], add=True)

        def _wait_old():
          pltpu.make_async_copy(rows.at[bg], acc.at[dbuf.at[c]],
                                ssem[bg]).wait()

        def _issue_gather():
          pltpu.async_copy(h_hbm.at[sbuf.at[c + _LOOK]], rows.at[bg],
                           gsem[bg])

        if b < _SKIP:
          @pl.when(t > 0)
          def _w():
            _wait_old()
          _issue_gather()
        else:
          _wait_old()

          @pl.when(t < _TRIPS - 1)
          def _g():
            _issue_gather()
      return carry
    lax.fori_loop(0, _TRIPS, _trip, 0)

    for b in range(_LOOK, _NBUF):
      pltpu.make_async_copy(rows.at[b], acc.at[dbuf.at[0]], ssem[b]).wait()

    @pl.when(wid < _REM)
    def _():
      pltpu.async_copy(h_hbm.at[sbuf.at[_FULL]], rows.at[0], gsem[0]).wait()
      pltpu.sync_copy(rows.at[0], acc.at[dbuf.at[_FULL]], add=True)

    plsc.subcore_barrier()
    pltpu.sync_copy(acc.at[pl.ds(sid * _SL, _SL)],
                    outp.at[cid, pl.ds(sid * _SL, _SL)])

  return pl.kernel(
      _agg_body,
      out_type=jax.ShapeDtypeStruct((_NC, _NPAD, d), jnp.float32),
      mesh=_mesh(),
      compiler_params=_SC_PARAMS,
      scratch_types=[
          pltpu.VMEM((_FULL + 1, _K), jnp.int32),
          pltpu.VMEM((_FULL + 1, _K), jnp.int32),
          pltpu.VMEM((_NBUF, _K, d), jnp.float32),
          pltpu.VMEM((32, d), jnp.float32),
          pltpu.VMEM_SHARED((_NPAD, d), jnp.float32),
      ] + [pltpu.SemaphoreType.DMA] * (2 * _NBUF),
  )


_agg64 = _make_agg(_HID, 6, 5)
_agg40 = _make_agg(_NCLS, 6, 5)


_BR = 1024
_GRID = (_N + _BR - 1) // _BR


def _mm1_body(x_ref, w_ref, degp_ref, o_ref):
  s = lax.rsqrt(degp_ref[0, :] + degp_ref[1, :] + 1.0)
  h = jnp.dot(x_ref[...], w_ref[...], preferred_element_type=jnp.float32)
  o_ref[...] = h * s[:, None]


def _mm1(x, w1, degp):
  return pl.pallas_call(
      _mm1_body,
      grid=(_GRID,),
      in_specs=[
          pl.BlockSpec((_BR, _FIN), lambda i: (i, 0)),
          pl.BlockSpec((_FIN, _HID), lambda i: (0, 0)),
          pl.BlockSpec((_NC, _BR), lambda i: (0, i)),
      ],
      out_specs=pl.BlockSpec((_BR, _HID), lambda i: (i, 0)),
      out_shape=jax.ShapeDtypeStruct((_NPAD, _HID), jnp.float32),
  )(x, w1, degp)


def _mm2_body(degp_ref, p_ref, h_ref, b_ref, w_ref, o_ref):
  s = lax.rsqrt(degp_ref[0, :] + degp_ref[1, :] + 1.0)
  t = (p_ref[0] + p_ref[1] + h_ref[...]) * s[:, None] + b_ref[...]
  z = jnp.maximum(t, 0.0)
  o_ref[...] = jnp.dot(
      z, w_ref[...], preferred_element_type=jnp.float32) * s[:, None]


def _mm2(degp, p, h1, b1r, w2):
  return pl.pallas_call(
      _mm2_body,
      grid=(_GRID,),
      in_specs=[
          pl.BlockSpec((_NC, _BR), lambda i: (0, i)),
          pl.BlockSpec((_NC, _BR, _HID), lambda i: (0, i, 0)),
          pl.BlockSpec((_BR, _HID), lambda i: (i, 0)),
          pl.BlockSpec((1, _HID), lambda i: (0, 0)),
          pl.BlockSpec((_HID, _NCLS), lambda i: (0, 0)),
      ],
      out_specs=pl.BlockSpec((_BR, _NCLS), lambda i: (i, 0)),
      out_shape=jax.ShapeDtypeStruct((_NPAD, _NCLS), jnp.float32),
  )(degp, p, h1, b1r, w2)


def _fin_body(degp_ref, p_ref, h_ref, b_ref, o_ref):
  s = lax.rsqrt(degp_ref[0, :] + degp_ref[1, :] + 1.0)
  u = (p_ref[0] + p_ref[1] + h_ref[...]) * s[:, None] + b_ref[...]
  m = jnp.max(u, axis=1, keepdims=True)
  lse = jnp.log(jnp.sum(jnp.exp(u - m), axis=1, keepdims=True)) + m
  o_ref[...] = u - lse


def _fin(degp, p, h2, b2r):
  return pl.pallas_call(
      _fin_body,
      grid=(_GRID,),
      in_specs=[
          pl.BlockSpec((_NC, _BR), lambda i: (0, i)),
          pl.BlockSpec((_NC, _BR, _NCLS), lambda i: (0, i, 0)),
          pl.BlockSpec((_BR, _NCLS), lambda i: (i, 0)),
          pl.BlockSpec((1, _NCLS), lambda i: (0, 0)),
      ],
      out_specs=pl.BlockSpec((_BR, _NCLS), lambda i: (i, 0)),
      out_shape=jax.ShapeDtypeStruct((_N, _NCLS), jnp.float32),
  )(degp, p, h2, b2r)


@jax.jit
def kernel(x, edge_index, W1, b1, W2, b2):
  ei3 = edge_index.astype(jnp.int32).reshape(2, _NCHUNK, _K)
  b1r = b1.reshape(1, _HID)
  b2r = b2.reshape(1, _NCLS)

  degp = _deg_call(ei3)
  h1 = _mm1(x, W1, degp)
  p1 = _agg64(h1, ei3)
  h2 = _mm2(degp, p1, h1, b1r, W2)
  p2 = _agg40(h2, ei3)
  return _fin(degp, p2, h2, b2r)


# speedup vs baseline: 1.0386x; 1.0358x over previous
import jax
import jax.numpy as jnp
from jax import lax
from jax.experimental import pallas as pl
from jax.experimental.pallas import tpu as pltpu
from jax.experimental.pallas import tpu_sc as plsc

_N = 10000
_E = 320000
_FIN = 128
_HID = 64
_NCLS = 40

_NC = 2
_NS = 16
_NW = _NC * _NS
_NPAD = 10240
_SL = _NPAD // _NS
_K = 128
_NCHUNK = _E // _K
_FULL = _NCHUNK // _NW
_REM = _NCHUNK - _FULL * _NW


def _mesh():
  return plsc.VectorSubcoreMesh(
      core_axis_name="c", subcore_axis_name="s", num_cores=_NC,
      num_subcores=_NS)


_SC_PARAMS = pltpu.CompilerParams(use_tc_tiling_on_sc=False)


_DK = 26


def _deg_body(ei_hbm, degp_hbm, dbuf, onesv, zbuf, acc, sem):
  cid = lax.axis_index("c")
  sid = lax.axis_index("s")
  wid = cid * _NS + sid

  pltpu.sync_copy(ei_hbm.at[1, pl.ds(wid * _FULL, _FULL)],
                  dbuf.at[pl.ds(0, _FULL)])

  @pl.when(wid < _REM)
  def _():
    pltpu.sync_copy(ei_hbm.at[1, pl.ds(_NW * _FULL + wid, 1)],
                    dbuf.at[pl.ds(_FULL, 1)])

  def _fill_z(i, c):
    zbuf[pl.ds(i * 16, 16)] = jnp.zeros((16,), jnp.float32)
    return c
  lax.fori_loop(0, _SL // 16, _fill_z, 0)

  def _fill_o(i, c):
    onesv[pl.ds(i * 16, 16)] = jnp.ones((16,), jnp.float32)
    return c
  lax.fori_loop(0, _K // 16, _fill_o, 0)

  pltpu.sync_copy(zbuf, acc.at[pl.ds(sid * _SL, _SL)])
  plsc.subcore_barrier()

  def _round(r, c):
    def _fire(j, c2):
      pltpu.async_copy(onesv, acc.at[dbuf.at[r * _DK + j]], sem, add=True)
      return c2
    lax.fori_loop(0, _DK, _fire, 0)

    def _drain(j, c2):
      pltpu.make_async_copy(onesv, acc.at[dbuf.at[r * _DK + j]], sem).wait()
      return c2
    lax.fori_loop(0, _DK, _drain, 0)
    return c
  lax.fori_loop(0, _FULL // _DK, _round, 0)

  @pl.when(wid < _REM)
  def _():
    pltpu.sync_copy(onesv, acc.at[dbuf.at[_FULL]], add=True)

  plsc.subcore_barrier()
  pltpu.sync_copy(acc.at[pl.ds(sid * _SL, _SL)],
                  degp_hbm.at[cid, pl.ds(sid * _SL, _SL)])


_deg_call = pl.kernel(
    _deg_body,
    out_type=jax.ShapeDtypeStruct((_NC, _NPAD), jnp.float32),
    mesh=_mesh(),
    compiler_params=_SC_PARAMS,
    scratch_types=[
        pltpu.VMEM((_FULL + 1, _K), jnp.int32),
        pltpu.VMEM((_K,), jnp.float32),
        pltpu.VMEM((_SL,), jnp.float32),
        pltpu.VMEM_SHARED((_NPAD,), jnp.float32),
        pltpu.SemaphoreType.DMA,
    ],
)


def _make_agg(d, _NBUF, _LOOK):
  _SKIP = _NBUF - _LOOK
  _TRIPS = _FULL // _NBUF

  def _agg_body(h_hbm, ei_hbm, outp, sbuf, dbuf, rows, zrows, acc, *sems):
    cid = lax.axis_index("c")
    sid = lax.axis_index("s")
    wid = cid * _NS + sid
    gsem = list(sems[:_NBUF])
    ssem = list(sems[_NBUF:])
    base = wid * _FULL

    ld_s = pltpu.async_copy(ei_hbm.at[0, pl.ds(base, _FULL)],
                            sbuf.at[pl.ds(0, _FULL)], gsem[0])
    ld_d = pltpu.async_copy(ei_hbm.at[1, pl.ds(base, _FULL)],
                            dbuf.at[pl.ds(0, _FULL)], gsem[1])

    @pl.when(wid < _REM)
    def _():
      pltpu.sync_copy(ei_hbm.at[0, pl.ds(_NW * _FULL + wid, 1)],
                      sbuf.at[pl.ds(_FULL, 1)])
      pltpu.sync_copy(ei_hbm.at[1, pl.ds(_NW * _FULL + wid, 1)],
                      dbuf.at[pl.ds(_FULL, 1)])

    offs = list(range(0, d - 15, 16))
    if d % 16 != 0:
      offs.append(d - 16)
    for i in range(32):
      for j in offs:
        zrows[i, pl.ds(j, 16)] = jnp.zeros((16,), jnp.float32)

    def _zero(t, c):
      pltpu.sync_copy(zrows, acc.at[pl.ds(sid * _SL + t * 32, 32)])
      return c
    lax.fori_loop(0, _SL // 32, _zero, 0)
    ld_s.wait()
    ld_d.wait()
    plsc.subcore_barrier()

    for b in range(_LOOK):
      pltpu.async_copy(h_hbm.at[sbuf.at[b]], rows.at[b], gsem[b])

    def _trip(t, carry):
      for b in range(_NBUF):
        c = t * _NBUF + b
        bg = (b + _LOOK) % _NBUF
        pltpu.make_async_copy(h_hbm.at[sbuf.at[c]], rows.at[b],
                              gsem[b]).wait()
        pltpu.async_copy(rows.at[b], acc.at[dbuf.at[c]], ssem[b], add=True)

        def _wait_old():
          pltpu.make_async_copy(rows.at[bg], acc.at[dbuf.at[c]],
                                ssem[bg]).wait()

        def _issue_gather():
          pltpu.async_copy(h_hbm.at[sbuf.at[c + _LOOK]], rows.at[bg],
                           gsem[bg])

        if b < _SKIP:
          @pl.when(t > 0)
          def _w():
            _wait_old()
          _issue_gather()
        else:
          _wait_old()

          @pl.when(t < _TRIPS - 1)
          def _g():
            _issue_gather()
      return carry
    lax.fori_loop(0, _TRIPS, _trip, 0)

    for b in range(_LOOK, _NBUF):
      pltpu.make_async_copy(rows.at[b], acc.at[dbuf.at[0]], ssem[b]).wait()

    @pl.when(wid < _REM)
    def _():
      pltpu.async_copy(h_hbm.at[sbuf.at[_FULL]], rows.at[0], gsem[0]).wait()
      pltpu.sync_copy(rows.at[0], acc.at[dbuf.at[_FULL]], add=True)

    plsc.subcore_barrier()
    pltpu.sync_copy(acc.at[pl.ds(sid * _SL, _SL)],
                    outp.at[cid, pl.ds(sid * _SL, _SL)])

  return pl.kernel(
      _agg_body,
      out_type=jax.ShapeDtypeStruct((_NC, _NPAD, d), jnp.float32),
      mesh=_mesh(),
      compiler_params=_SC_PARAMS,
      scratch_types=[
          pltpu.VMEM((_FULL + 1, _K), jnp.int32),
          pltpu.VMEM((_FULL + 1, _K), jnp.int32),
          pltpu.VMEM((_NBUF, _K, d), jnp.float32),
          pltpu.VMEM((32, d), jnp.float32),
          pltpu.VMEM_SHARED((_NPAD, d), jnp.float32),
      ] + [pltpu.SemaphoreType.DMA] * (2 * _NBUF),
  )


_agg64 = _make_agg(_HID, 6, 5)
_agg40 = _make_agg(_NCLS, 6, 5)


_BR = 2048
_GRID = (_N + _BR - 1) // _BR


def _mm1_body(x_ref, w_ref, degp_ref, o_ref):
  s = lax.rsqrt(degp_ref[0, :] + degp_ref[1, :] + 1.0)
  h = jnp.dot(x_ref[...], w_ref[...], preferred_element_type=jnp.float32)
  o_ref[...] = h * s[:, None]


def _mm1(x, w1, degp):
  return pl.pallas_call(
      _mm1_body,
      grid=(_GRID,),
      in_specs=[
          pl.BlockSpec((_BR, _FIN), lambda i: (i, 0)),
          pl.BlockSpec((_FIN, _HID), lambda i: (0, 0)),
          pl.BlockSpec((_NC, _BR), lambda i: (0, i)),
      ],
      out_specs=pl.BlockSpec((_BR, _HID), lambda i: (i, 0)),
      out_shape=jax.ShapeDtypeStruct((_NPAD, _HID), jnp.float32),
  )(x, w1, degp)


def _mm2_body(degp_ref, p_ref, h_ref, b_ref, w_ref, o_ref):
  s = lax.rsqrt(degp_ref[0, :] + degp_ref[1, :] + 1.0)
  t = (p_ref[0] + p_ref[1] + h_ref[...]) * s[:, None] + b_ref[...]
  z = jnp.maximum(t, 0.0)
  o_ref[...] = jnp.dot(
      z, w_ref[...], preferred_element_type=jnp.float32) * s[:, None]


def _mm2(degp, p, h1, b1r, w2):
  return pl.pallas_call(
      _mm2_body,
      grid=(_GRID,),
      in_specs=[
          pl.BlockSpec((_NC, _BR), lambda i: (0, i)),
          pl.BlockSpec((_NC, _BR, _HID), lambda i: (0, i, 0)),
          pl.BlockSpec((_BR, _HID), lambda i: (i, 0)),
          pl.BlockSpec((1, _HID), lambda i: (0, 0)),
          pl.BlockSpec((_HID, _NCLS), lambda i: (0, 0)),
      ],
      out_specs=pl.BlockSpec((_BR, _NCLS), lambda i: (i, 0)),
      out_shape=jax.ShapeDtypeStruct((_NPAD, _NCLS), jnp.float32),
  )(degp, p, h1, b1r, w2)


def _fin_body(degp_ref, p_ref, h_ref, b_ref, o_ref):
  s = lax.rsqrt(degp_ref[0, :] + degp_ref[1, :] + 1.0)
  u = (p_ref[0] + p_ref[1] + h_ref[...]) * s[:, None] + b_ref[...]
  m = jnp.max(u, axis=1, keepdims=True)
  lse = jnp.log(jnp.sum(jnp.exp(u - m), axis=1, keepdims=True)) + m
  o_ref[...] = u - lse


def _fin(degp, p, h2, b2r):
  return pl.pallas_call(
      _fin_body,
      grid=(_GRID,),
      in_specs=[
          pl.BlockSpec((_NC, _BR), lambda i: (0, i)),
          pl.BlockSpec((_NC, _BR, _NCLS), lambda i: (0, i, 0)),
          pl.BlockSpec((_BR, _NCLS), lambda i: (i, 0)),
          pl.BlockSpec((1, _NCLS), lambda i: (0, 0)),
      ],
      out_specs=pl.BlockSpec((_BR, _NCLS), lambda i: (i, 0)),
      out_shape=jax.ShapeDtypeStruct((_N, _NCLS), jnp.float32),
  )(degp, p, h2, b2r)


@jax.jit
def kernel(x, edge_index, W1, b1, W2, b2):
  ei3 = edge_index.astype(jnp.int32).reshape(2, _NCHUNK, _K)
  b1r = b1.reshape(1, _HID)
  b2r = b2.reshape(1, _NCLS)

  degp = _deg_call(ei3)
  h1 = _mm1(x, W1, degp)
  p1 = _agg64(h1, ei3)
  h2 = _mm2(degp, p1, h1, b1r, W2)
  p2 = _agg40(h2, ei3)
  return _fin(degp, p2, h2, b2r)


# speedup vs baseline: 1.0699x; 1.0301x over previous
import jax
import jax.numpy as jnp
from jax import lax
from jax.experimental import pallas as pl
from jax.experimental.pallas import tpu as pltpu
from jax.experimental.pallas import tpu_sc as plsc

_N = 10000
_E = 320000
_FIN = 128
_HID = 64
_NCLS = 40

_NC = 2
_NS = 16
_NW = _NC * _NS
_NPAD = 10240
_SL = _NPAD // _NS
_K = 128
_NCHUNK = _E // _K
_FULL = _NCHUNK // _NW
_REM = _NCHUNK - _FULL * _NW


def _mesh():
  return plsc.VectorSubcoreMesh(
      core_axis_name="c", subcore_axis_name="s", num_cores=_NC,
      num_subcores=_NS)


_SC_PARAMS = pltpu.CompilerParams(use_tc_tiling_on_sc=False)


_DK = 26


def _deg_body(ei_hbm, degp_hbm, dbuf, onesv, zbuf, acc, sem):
  cid = lax.axis_index("c")
  sid = lax.axis_index("s")
  wid = cid * _NS + sid

  pltpu.sync_copy(ei_hbm.at[1, pl.ds(wid * _FULL, _FULL)],
                  dbuf.at[pl.ds(0, _FULL)])

  @pl.when(wid < _REM)
  def _():
    pltpu.sync_copy(ei_hbm.at[1, pl.ds(_NW * _FULL + wid, 1)],
                    dbuf.at[pl.ds(_FULL, 1)])

  def _fill_z(i, c):
    zbuf[pl.ds(i * 16, 16)] = jnp.zeros((16,), jnp.float32)
    return c
  lax.fori_loop(0, _SL // 16, _fill_z, 0)

  def _fill_o(i, c):
    onesv[pl.ds(i * 16, 16)] = jnp.ones((16,), jnp.float32)
    return c
  lax.fori_loop(0, _K // 16, _fill_o, 0)

  pltpu.sync_copy(zbuf, acc.at[pl.ds(sid * _SL, _SL)])
  plsc.subcore_barrier()

  def _round(r, c):
    def _fire(j, c2):
      pltpu.async_copy(onesv, acc.at[dbuf.at[r * _DK + j]], sem, add=True)
      return c2
    lax.fori_loop(0, _DK, _fire, 0)

    def _drain(j, c2):
      pltpu.make_async_copy(onesv, acc.at[dbuf.at[r * _DK + j]], sem).wait()
      return c2
    lax.fori_loop(0, _DK, _drain, 0)
    return c
  lax.fori_loop(0, _FULL // _DK, _round, 0)

  @pl.when(wid < _REM)
  def _():
    pltpu.sync_copy(onesv, acc.at[dbuf.at[_FULL]], add=True)

  plsc.subcore_barrier()
  pltpu.sync_copy(acc.at[pl.ds(sid * _SL, _SL)],
                  degp_hbm.at[cid, pl.ds(sid * _SL, _SL)])


_deg_call = pl.kernel(
    _deg_body,
    out_type=jax.ShapeDtypeStruct((_NC, _NPAD), jnp.float32),
    mesh=_mesh(),
    compiler_params=_SC_PARAMS,
    scratch_types=[
        pltpu.VMEM((_FULL + 1, _K), jnp.int32),
        pltpu.VMEM((_K,), jnp.float32),
        pltpu.VMEM((_SL,), jnp.float32),
        pltpu.VMEM_SHARED((_NPAD,), jnp.float32),
        pltpu.SemaphoreType.DMA,
    ],
)


def _make_agg(d, _NBUF, _LOOK):
  _SKIP = _NBUF - _LOOK
  _TRIPS = _FULL // _NBUF

  def _agg_body(h_hbm, ei_hbm, outp, sbuf, dbuf, rows, zrows, acc, *sems):
    cid = lax.axis_index("c")
    sid = lax.axis_index("s")
    wid = cid * _NS + sid
    gsem = list(sems[:_NBUF])
    ssem = list(sems[_NBUF:])
    base = wid * _FULL

    ld_s = pltpu.async_copy(ei_hbm.at[0, pl.ds(base, _FULL)],
                            sbuf.at[pl.ds(0, _FULL)], gsem[0])
    ld_d = pltpu.async_copy(ei_hbm.at[1, pl.ds(base, _FULL)],
                            dbuf.at[pl.ds(0, _FULL)], gsem[1])

    @pl.when(wid < _REM)
    def _():
      pltpu.sync_copy(ei_hbm.at[0, pl.ds(_NW * _FULL + wid, 1)],
                      sbuf.at[pl.ds(_FULL, 1)])
      pltpu.sync_copy(ei_hbm.at[1, pl.ds(_NW * _FULL + wid, 1)],
                      dbuf.at[pl.ds(_FULL, 1)])

    offs = list(range(0, d - 15, 16))
    if d % 16 != 0:
      offs.append(d - 16)
    for i in range(32):
      for j in offs:
        zrows[i, pl.ds(j, 16)] = jnp.zeros((16,), jnp.float32)

    def _zero(t, c):
      pltpu.sync_copy(zrows, acc.at[pl.ds(sid * _SL + t * 32, 32)])
      return c
    lax.fori_loop(0, _SL // 32, _zero, 0)
    ld_s.wait()
    ld_d.wait()
    plsc.subcore_barrier()

    for b in range(_LOOK):
      pltpu.async_copy(h_hbm.at[sbuf.at[b]], rows.at[b], gsem[b])

    def _trip(t, carry):
      for b in range(_NBUF):
        c = t * _NBUF + b
        bg = (b + _LOOK) % _NBUF
        pltpu.make_async_copy(h_hbm.at[sbuf.at[c]], rows.at[b],
                              gsem[b]).wait()
        pltpu.async_copy(rows.at[b], acc.at[dbuf.at[c]], ssem[b], add=True)

        def _wait_old():
          pltpu.make_async_copy(rows.at[bg], acc.at[dbuf.at[c]],
                                ssem[bg]).wait()

        def _issue_gather():
          pltpu.async_copy(h_hbm.at[sbuf.at[c + _LOOK]], rows.at[bg],
                           gsem[bg])

        if b < _SKIP:
          @pl.when(t > 0)
          def _w():
            _wait_old()
          _issue_gather()
        else:
          _wait_old()

          @pl.when(t < _TRIPS - 1)
          def _g():
            _issue_gather()
      return carry
    lax.fori_loop(0, _TRIPS, _trip, 0)

    for b in range(_LOOK, _NBUF):
      pltpu.make_async_copy(rows.at[b], acc.at[dbuf.at[0]], ssem[b]).wait()

    @pl.when(wid < _REM)
    def _():
      pltpu.async_copy(h_hbm.at[sbuf.at[_FULL]], rows.at[0], gsem[0]).wait()
      pltpu.sync_copy(rows.at[0], acc.at[dbuf.at[_FULL]], add=True)

    plsc.subcore_barrier()
    pltpu.sync_copy(acc.at[pl.ds(sid * _SL, _SL)],
                    outp.at[cid, pl.ds(sid * _SL, _SL)])

  return pl.kernel(
      _agg_body,
      out_type=jax.ShapeDtypeStruct((_NC, _NPAD, d), jnp.float32),
      mesh=_mesh(),
      compiler_params=_SC_PARAMS,
      scratch_types=[
          pltpu.VMEM((_FULL + 1, _K), jnp.int32),
          pltpu.VMEM((_FULL + 1, _K), jnp.int32),
          pltpu.VMEM((_NBUF, _K, d), jnp.float32),
          pltpu.VMEM((32, d), jnp.float32),
          pltpu.VMEM_SHARED((_NPAD, d), jnp.float32),
      ] + [pltpu.SemaphoreType.DMA] * (2 * _NBUF),
  )


_agg64 = _make_agg(_HID, 6, 5)
_agg40 = _make_agg(_NCLS, 6, 5)


_BR = 5120
_GRID = (_N + _BR - 1) // _BR


def _mm1_body(x_ref, w_ref, degp_ref, o_ref):
  s = lax.rsqrt(degp_ref[0, :] + degp_ref[1, :] + 1.0)
  h = jnp.dot(x_ref[...], w_ref[...], preferred_element_type=jnp.float32)
  o_ref[...] = h * s[:, None]


def _mm1(x, w1, degp):
  return pl.pallas_call(
      _mm1_body,
      grid=(_GRID,),
      in_specs=[
          pl.BlockSpec((_BR, _FIN), lambda i: (i, 0)),
          pl.BlockSpec((_FIN, _HID), lambda i: (0, 0)),
          pl.BlockSpec((_NC, _BR), lambda i: (0, i)),
      ],
      out_specs=pl.BlockSpec((_BR, _HID), lambda i: (i, 0)),
      out_shape=jax.ShapeDtypeStruct((_NPAD, _HID), jnp.float32),
  )(x, w1, degp)


def _mm2_body(degp_ref, p_ref, h_ref, b_ref, w_ref, o_ref):
  s = lax.rsqrt(degp_ref[0, :] + degp_ref[1, :] + 1.0)
  t = (p_ref[0] + p_ref[1] + h_ref[...]) * s[:, None] + b_ref[...]
  z = jnp.maximum(t, 0.0)
  o_ref[...] = jnp.dot(
      z, w_ref[...], preferred_element_type=jnp.float32) * s[:, None]


def _mm2(degp, p, h1, b1r, w2):
  return pl.pallas_call(
      _mm2_body,
      grid=(_GRID,),
      in_specs=[
          pl.BlockSpec((_NC, _BR), lambda i: (0, i)),
          pl.BlockSpec((_NC, _BR, _HID), lambda i: (0, i, 0)),
          pl.BlockSpec((_BR, _HID), lambda i: (i, 0)),
          pl.BlockSpec((1, _HID), lambda i: (0, 0)),
          pl.BlockSpec((_HID, _NCLS), lambda i: (0, 0)),
      ],
      out_specs=pl.BlockSpec((_BR, _NCLS), lambda i: (i, 0)),
      out_shape=jax.ShapeDtypeStruct((_NPAD, _NCLS), jnp.float32),
  )(degp, p, h1, b1r, w2)


def _fin_body(degp_ref, p_ref, h_ref, b_ref, o_ref):
  s = lax.rsqrt(degp_ref[0, :] + degp_ref[1, :] + 1.0)
  u = (p_ref[0] + p_ref[1] + h_ref[...]) * s[:, None] + b_ref[...]
  m = jnp.max(u, axis=1, keepdims=True)
  lse = jnp.log(jnp.sum(jnp.exp(u - m), axis=1, keepdims=True)) + m
  o_ref[...] = u - lse


def _fin(degp, p, h2, b2r):
  return pl.pallas_call(
      _fin_body,
      grid=(_GRID,),
      in_specs=[
          pl.BlockSpec((_NC, _BR), lambda i: (0, i)),
          pl.BlockSpec((_NC, _BR, _NCLS), lambda i: (0, i, 0)),
          pl.BlockSpec((_BR, _NCLS), lambda i: (i, 0)),
          pl.BlockSpec((1, _NCLS), lambda i: (0, 0)),
      ],
      out_specs=pl.BlockSpec((_BR, _NCLS), lambda i: (i, 0)),
      out_shape=jax.ShapeDtypeStruct((_N, _NCLS), jnp.float32),
  )(degp, p, h2, b2r)


@jax.jit
def kernel(x, edge_index, W1, b1, W2, b2):
  ei3 = edge_index.astype(jnp.int32).reshape(2, _NCHUNK, _K)
  b1r = b1.reshape(1, _HID)
  b2r = b2.reshape(1, _NCLS)

  degp = _deg_call(ei3)
  h1 = _mm1(x, W1, degp)
  p1 = _agg64(h1, ei3)
  h2 = _mm2(degp, p1, h1, b1r, W2)
  p2 = _agg40(h2, ei3)
  return _fin(degp, p2, h2, b2r)
